# baseline (speedup 1.0000x reference)
import functools

import jax
import jax.numpy as jnp
from jax import lax
from jax.experimental import pallas as pl
from jax.experimental.pallas import tpu as pltpu
from jax.experimental.pallas import tpu_sc as plsc

B = 16384
H = 50
D = 32
BH = B * H
NC = 2
NS = 16
NW = NC * NS
L = BH // NW
CH = 128
NCH = L // CH
S = 10
N_ROUNDS = NCH // S

_mesh = plsc.VectorSubcoreMesh(core_axis_name="c", subcore_axis_name="s")


@functools.partial(
    pl.kernel,
    out_type=jax.ShapeDtypeStruct((BH, D), jnp.float32),
    mesh=_mesh,
    compiler_params=pltpu.CompilerParams(use_tc_tiling_on_sc=False),
    scratch_types=[
        pltpu.VMEM((L,), jnp.int32),
        pltpu.VMEM((S, 2, CH, D), jnp.float32),
        pltpu.SemaphoreType.DMA((S,)),
        pltpu.SemaphoreType.DMA((S,)),
    ],
)
def _gather(idx_hbm, table_hbm, out_hbm, idx_v, buf, gsem, ssem):
    wid = lax.axis_index("s") * NC + lax.axis_index("c")
    base = wid * L

    pltpu.sync_copy(idx_hbm.at[pl.ds(base, L)], idx_v)

    def fire(c, half, s):
        pltpu.async_copy(
            table_hbm.at[idx_v.at[pl.ds(c * CH, CH)]], buf.at[s].at[half],
            gsem.at[s],
        )

    def wait_gather(c, half, s):
        pltpu.make_async_copy(
            table_hbm.at[idx_v.at[pl.ds(c * CH, CH)]], buf.at[s].at[half],
            gsem.at[s],
        ).wait()

    def store(c, half, s):
        pltpu.async_copy(
            buf.at[s].at[half], out_hbm.at[pl.ds(base + c * CH, CH)],
            ssem.at[s],
        )

    def drain_one_store(half, s):
        pltpu.make_async_copy(
            table_hbm.at[pl.ds(0, CH)], buf.at[s].at[half], ssem.at[s]
        ).wait()

    for s in range(S):
        fire(s, 0, s)

    for s in range(S):
        wait_gather(s, 0, s)
        store(s, 0, s)
        fire(s + S, 1, s)

    def round_body(r, carry):
        h = r % 2
        hn = 1 - h
        for s in range(S):
            j = r * S + s
            wait_gather(j, h, s)
            store(j, h, s)
            drain_one_store(hn, s)
            fire(j + S, hn, s)
        return carry

    lax.fori_loop(1, N_ROUNDS - 1, round_body, 0)

    hl = (N_ROUNDS - 1) % 2
    for s in range(S):
        j = (N_ROUNDS - 1) * S + s
        wait_gather(j, hl, s)
        store(j, hl, s)

    for s in range(S):
        drain_one_store(0, s)
        drain_one_store(1, s)


def kernel(input, weight):
    idx = input.astype(jnp.int32).reshape(BH)
    out = _gather(idx, weight)
    return out.reshape(B, H, D)


# speedup vs baseline: 1.0004x; 1.0004x over previous
import functools

import jax
import jax.numpy as jnp
from jax import lax
from jax.experimental import pallas as pl
from jax.experimental.pallas import tpu as pltpu
from jax.experimental.pallas import tpu_sc as plsc

B = 16384
H = 50
D = 32
BH = B * H
NC = 2
NS = 16
NW = NC * NS
L = BH // NW
CH = 128
NCH = L // CH
S = 8
N_ROUNDS = NCH // S

_mesh = plsc.VectorSubcoreMesh(core_axis_name="c", subcore_axis_name="s")


@functools.partial(
    pl.kernel,
    out_type=jax.ShapeDtypeStruct((BH, D), jnp.float32),
    mesh=_mesh,
    compiler_params=pltpu.CompilerParams(use_tc_tiling_on_sc=False),
    scratch_types=[
        pltpu.VMEM((L,), jnp.int32),
        pltpu.VMEM((S, 2, CH, D), jnp.float32),
        pltpu.SemaphoreType.DMA((S,)),
        pltpu.SemaphoreType.DMA((S,)),
    ],
)
def _gather(idx_hbm, table_hbm, out_hbm, idx_v, buf, gsem, ssem):
    wid = lax.axis_index("s") * NC + lax.axis_index("c")
    base = wid * L

    pltpu.sync_copy(idx_hbm.at[pl.ds(base, L)], idx_v)

    def fire(c, half, s):
        pltpu.async_copy(
            table_hbm.at[idx_v.at[pl.ds(c * CH, CH)]], buf.at[s].at[half],
            gsem.at[s],
        )

    def wait_gather(c, half, s):
        pltpu.make_async_copy(
            table_hbm.at[idx_v.at[pl.ds(c * CH, CH)]], buf.at[s].at[half],
            gsem.at[s],
        ).wait()

    def store(c, half, s):
        pltpu.async_copy(
            buf.at[s].at[half], out_hbm.at[pl.ds(base + c * CH, CH)],
            ssem.at[s],
        )

    def drain_one_store(half, s):
        pltpu.make_async_copy(
            table_hbm.at[pl.ds(0, CH)], buf.at[s].at[half], ssem.at[s]
        ).wait()

    for s in range(S):
        fire(s, 0, s)

    for s in range(S):
        wait_gather(s, 0, s)
        store(s, 0, s)
        fire(s + S, 1, s)

    def round_body(r, carry):
        h = r % 2
        hn = 1 - h
        for s in range(S):
            j = r * S + s
            wait_gather(j, h, s)
            store(j, h, s)
            drain_one_store(hn, s)
            fire(j + S, hn, s)
        return carry

    lax.fori_loop(1, N_ROUNDS - 1, round_body, 0)

    hl = (N_ROUNDS - 1) % 2
    for s in range(S):
        j = (N_ROUNDS - 1) * S + s
        wait_gather(j, hl, s)
        store(j, hl, s)

    for s in range(S):
        drain_one_store(0, s)
        drain_one_store(1, s)


def kernel(input, weight):
    idx = input.astype(jnp.int32).reshape(BH)
    out = _gather(idx, weight)
    return out.reshape(B, H, D)


# speedup vs baseline: 1.6168x; 1.6161x over previous
import functools

import jax
import jax.numpy as jnp
from jax import lax
from jax.experimental import pallas as pl
from jax.experimental.pallas import tpu as pltpu
from jax.experimental.pallas import tpu_sc as plsc

B = 16384
H = 50
D = 32
NC = 2
NS = 16
NW = NC * NS
ROWS_W = B // NW
S = 8
N_ROUNDS = ROWS_W // S

_mesh = plsc.VectorSubcoreMesh(core_axis_name="c", subcore_axis_name="s")


@functools.partial(
    pl.kernel,
    out_type=jax.ShapeDtypeStruct((B, H, D), jnp.float32),
    mesh=_mesh,
    compiler_params=pltpu.CompilerParams(use_tc_tiling_on_sc=False),
    scratch_types=[
        pltpu.VMEM((ROWS_W, H), jnp.int32),
        pltpu.VMEM((S, 2, 1, H, D), jnp.float32),
        pltpu.SemaphoreType.DMA((S,)),
        pltpu.SemaphoreType.DMA((S,)),
    ],
)
def _gather(idx_hbm, table_hbm, out_hbm, idx_v, buf, gsem, ssem):
    wid = lax.axis_index("s") * NC + lax.axis_index("c")
    base = wid * ROWS_W

    pltpu.sync_copy(idx_hbm.at[pl.ds(base, ROWS_W)], idx_v)

    def fire(r, half, s):
        pltpu.async_copy(
            table_hbm.at[idx_v.at[r]], buf.at[s].at[half].at[0], gsem.at[s]
        )

    def wait_gather(r, half, s):
        pltpu.make_async_copy(
            table_hbm.at[idx_v.at[r]], buf.at[s].at[half].at[0], gsem.at[s]
        ).wait()

    def store(r, half, s):
        pltpu.async_copy(
            buf.at[s].at[half], out_hbm.at[pl.ds(base + r, 1)], ssem.at[s]
        )

    def drain_one_store(half, s):
        pltpu.make_async_copy(
            table_hbm.at[pl.ds(0, H)], buf.at[s].at[half].at[0], ssem.at[s]
        ).wait()

    for s in range(S):
        fire(s, 0, s)

    for s in range(S):
        wait_gather(s, 0, s)
        store(s, 0, s)
        fire(s + S, 1, s)

    def round_body(r, carry):
        h = r % 2
        hn = 1 - h
        for s in range(S):
            j = r * S + s
            wait_gather(j, h, s)
            store(j, h, s)
            drain_one_store(hn, s)
            fire(j + S, hn, s)
        return carry

    lax.fori_loop(1, N_ROUNDS - 1, round_body, 0)

    hl = (N_ROUNDS - 1) % 2
    for s in range(S):
        j = (N_ROUNDS - 1) * S + s
        wait_gather(j, hl, s)
        store(j, hl, s)

    for s in range(S):
        drain_one_store(0, s)
        drain_one_store(1, s)


def kernel(input, weight):
    return _gather(input.astype(jnp.int32), weight)
